# Initial kernel scaffold; baseline (speedup 1.0000x reference)
import jax
import jax.numpy as jnp
from jax.experimental import pallas as pl


def kernel(input, adj, h0, weight, w_att, lamda, alpha, l):
    raise NotImplementedError("write your pallas kernel here")


# speedup vs baseline: 1.6329x; 1.6329x over previous
import math

import jax
import jax.numpy as jnp
from jax.experimental import pallas as pl
from jax.experimental.pallas import tpu as pltpu

_MOMENT = 4
_BM = 256
_BK = 1024


def _cdiv(a, b):
    return -(-a // b)


def kernel(input, adj, h0, weight, w_att, lamda, alpha, l):
    n, d = input.shape
    nd = _MOMENT + 1
    alpha = jnp.asarray(alpha, jnp.float32)

    h0_2 = h0 * h0
    x = jnp.concatenate(
        [(1.0 - alpha) * input, h0, h0_2, h0_2 * h0, h0_2 * h0_2], axis=1
    )
    kpad = _cdiv(n, _BK) * _BK
    x = jnp.pad(x, ((0, kpad - n), (0, 0)))
    h0a = alpha * h0

    theta = math.log(1.5)
    beta = 0.9
    nk = kpad // _BK

    def body(adj_ref, x_ref, h0a_ref, w_ref, watt_ref, out_ref, acc_ref):
        k = pl.program_id(1)

        @pl.when(k == 0)
        def _init():
            acc_ref[...] = jnp.zeros_like(acc_ref)

        col = k * _BK + jax.lax.broadcasted_iota(jnp.int32, (1, _BK), 1)
        adj_blk = jnp.where(col < n, adj_ref[...], 0.0)
        x_blk = x_ref[pl.ds(k * _BK, _BK), :]
        acc_ref[...] += jnp.dot(adj_blk, x_blk,
                                preferred_element_type=jnp.float32)

        @pl.when(k == nk - 1)
        def _epilogue():
            p = acc_ref[...]
            h_agg = p[:, 0:d] + h0a_ref[...]
            h_i = theta * jnp.dot(h_agg, w_ref[...],
                                  preferred_element_type=jnp.float32)
            h_i = h_i + (1.0 - theta) * h_agg

            mu = p[:, d:2 * d]
            s = p[:, 2 * d:3 * d]
            s = jnp.where(s == 0.0, 1e-16, s)
            sig = jnp.sqrt(s)
            g3 = p[:, 3 * d:4 * d]
            g3 = jnp.where(g3 == 0.0, 1e-16, g3)
            a3 = jnp.abs(g3) ** (1.0 / 3.0)
            m3 = jnp.where(g3 < 0, -a3, a3)
            g4 = p[:, 4 * d:5 * d]
            g4 = jnp.where(g4 == 0.0, 1e-16, g4)
            a4 = jnp.abs(g4) ** 0.25
            m4 = jnp.where(g4 < 0, -a4, a4)

            wt = watt_ref[0:d, :]
            wb = watt_ref[d:2 * d, :]
            hw = jnp.dot(h_i, wb, preferred_element_type=jnp.float32)
            moms = (mu, sig, m3, m4)
            es = [
                jnp.dot(m, wt, preferred_element_type=jnp.float32) + hw
                for m in moms
            ]
            es = [jnp.where(e > 0, e, jnp.exp(e) - 1.0) for e in es]
            emax = jnp.maximum(jnp.maximum(es[0], es[1]),
                               jnp.maximum(es[2], es[3]))
            ws = [jnp.exp(e - emax) for e in es]
            denom = ws[0] + ws[1] + ws[2] + ws[3]
            h_m = (moms[0] * ws[0] + moms[1] * ws[1]
                   + moms[2] * ws[2] + moms[3] * ws[3]) / denom
            out_ref[...] = (1.0 - beta) * h_i + beta * h_m

    grid = (_cdiv(n, _BM), nk)
    out = pl.pallas_call(
        body,
        grid=grid,
        in_specs=[
            pl.BlockSpec((_BM, _BK), lambda i, k: (i, k)),
            pl.BlockSpec((kpad, nd * d), lambda i, k: (0, 0)),
            pl.BlockSpec((_BM, d), lambda i, k: (i, 0)),
            pl.BlockSpec((d, d), lambda i, k: (0, 0)),
            pl.BlockSpec((2 * d, d), lambda i, k: (0, 0)),
        ],
        out_specs=pl.BlockSpec((_BM, d), lambda i, k: (i, 0)),
        out_shape=jax.ShapeDtypeStruct((n, d), jnp.float32),
        scratch_shapes=[pltpu.VMEM((_BM, nd * d), jnp.float32)],
        compiler_params=pltpu.CompilerParams(
            dimension_semantics=("parallel", "arbitrary")),
    )(adj, x, h0a, weight, w_att)
    return out


# speedup vs baseline: 1.6598x; 1.0164x over previous
import math

import jax
import jax.numpy as jnp
from jax.experimental import pallas as pl
from jax.experimental.pallas import tpu as pltpu

_MOMENT = 4
_BM = 256
_BK = 1024


def _cdiv(a, b):
    return -(-a // b)


def kernel(input, adj, h0, weight, w_att, lamda, alpha, l):
    n, d = input.shape
    nd = _MOMENT + 1
    alpha = jnp.asarray(alpha, jnp.float32)

    h0_2 = h0 * h0
    x = jnp.concatenate(
        [(1.0 - alpha) * input, h0, h0_2, h0_2 * h0, h0_2 * h0_2], axis=1
    )
    kpad = _cdiv(n, _BK) * _BK
    x = jnp.pad(x, ((0, kpad - n), (0, 0))).astype(jnp.bfloat16)
    h0a = alpha * h0

    theta = math.log(1.5)
    beta = 0.9
    nk = kpad // _BK

    def body(adj_ref, x_ref, h0a_ref, w_ref, watt_ref, out_ref, acc_ref):
        k = pl.program_id(1)

        @pl.when(k == 0)
        def _init():
            acc_ref[...] = jnp.zeros_like(acc_ref)

        col = k * _BK + jax.lax.broadcasted_iota(jnp.int32, (1, _BK), 1)
        adj_blk = jnp.where(col < n, adj_ref[...], 0.0).astype(jnp.bfloat16)
        x_blk = x_ref[pl.ds(k * _BK, _BK), :]
        acc_ref[...] += jnp.dot(adj_blk, x_blk,
                                preferred_element_type=jnp.float32)

        @pl.when(k == nk - 1)
        def _epilogue():
            p = acc_ref[...]
            h_agg = p[:, 0:d] + h0a_ref[...]
            h_i = theta * jnp.dot(h_agg, w_ref[...],
                                  preferred_element_type=jnp.float32)
            h_i = h_i + (1.0 - theta) * h_agg

            mu = p[:, d:2 * d]
            s = p[:, 2 * d:3 * d]
            s = jnp.where(s == 0.0, 1e-16, s)
            sig = jnp.sqrt(s)
            g3 = p[:, 3 * d:4 * d]
            g3 = jnp.where(g3 == 0.0, 1e-16, g3)
            a3 = jnp.abs(g3) ** (1.0 / 3.0)
            m3 = jnp.where(g3 < 0, -a3, a3)
            g4 = p[:, 4 * d:5 * d]
            g4 = jnp.where(g4 == 0.0, 1e-16, g4)
            a4 = jnp.abs(g4) ** 0.25
            m4 = jnp.where(g4 < 0, -a4, a4)

            wt = watt_ref[0:d, :]
            wb = watt_ref[d:2 * d, :]
            hw = jnp.dot(h_i, wb, preferred_element_type=jnp.float32)
            moms = (mu, sig, m3, m4)
            es = [
                jnp.dot(m, wt, preferred_element_type=jnp.float32) + hw
                for m in moms
            ]
            es = [jnp.where(e > 0, e, jnp.exp(e) - 1.0) for e in es]
            emax = jnp.maximum(jnp.maximum(es[0], es[1]),
                               jnp.maximum(es[2], es[3]))
            ws = [jnp.exp(e - emax) for e in es]
            denom = ws[0] + ws[1] + ws[2] + ws[3]
            h_m = (moms[0] * ws[0] + moms[1] * ws[1]
                   + moms[2] * ws[2] + moms[3] * ws[3]) / denom
            out_ref[...] = (1.0 - beta) * h_i + beta * h_m

    grid = (_cdiv(n, _BM), nk)
    out = pl.pallas_call(
        body,
        grid=grid,
        in_specs=[
            pl.BlockSpec((_BM, _BK), lambda i, k: (i, k)),
            pl.BlockSpec((kpad, nd * d), lambda i, k: (0, 0)),
            pl.BlockSpec((_BM, d), lambda i, k: (i, 0)),
            pl.BlockSpec((d, d), lambda i, k: (0, 0)),
            pl.BlockSpec((2 * d, d), lambda i, k: (0, 0)),
        ],
        out_specs=pl.BlockSpec((_BM, d), lambda i, k: (i, 0)),
        out_shape=jax.ShapeDtypeStruct((n, d), jnp.float32),
        scratch_shapes=[pltpu.VMEM((_BM, nd * d), jnp.float32)],
        compiler_params=pltpu.CompilerParams(
            dimension_semantics=("parallel", "arbitrary")),
    )(adj, x, h0a, weight, w_att)
    return out


# speedup vs baseline: 2.4501x; 1.4762x over previous
import math

import jax
import jax.numpy as jnp
from jax.experimental import pallas as pl
from jax.experimental.pallas import tpu as pltpu

_MOMENT = 4
_BM = 512
_BK = 2048


def _cdiv(a, b):
    return -(-a // b)


def kernel(input, adj, h0, weight, w_att, lamda, alpha, l):
    n, d = input.shape
    nd = _MOMENT + 1
    alpha = jnp.asarray(alpha, jnp.float32)

    h0_2 = h0 * h0
    x = jnp.concatenate(
        [(1.0 - alpha) * input, h0, h0_2, h0_2 * h0, h0_2 * h0_2], axis=1
    )
    kpad = _cdiv(n, _BK) * _BK
    x = jnp.pad(x, ((0, kpad - n), (0, 0))).astype(jnp.bfloat16)
    h0a = alpha * h0

    theta = math.log(1.5)
    beta = 0.9
    nk = kpad // _BK
    tail = n - (nk - 1) * _BK

    def body(adj_ref, x_ref, h0a_ref, w_ref, watt_ref, out_ref, acc_ref):
        k = pl.program_id(1)

        @pl.when(k == 0)
        def _init():
            acc_ref[...] = jnp.zeros_like(acc_ref)

        def full_step():
            a = adj_ref[...].astype(jnp.bfloat16)
            xb = x_ref[pl.ds(k * _BK, _BK), :]
            return jnp.dot(a, xb, preferred_element_type=jnp.float32)

        def tail_step():
            a = adj_ref[:, 0:tail].astype(jnp.bfloat16)
            xb = x_ref[pl.ds(k * _BK, tail), :]
            return jnp.dot(a, xb, preferred_element_type=jnp.float32)

        acc_ref[...] += jax.lax.cond(k == nk - 1, tail_step, full_step)

        @pl.when(k == nk - 1)
        def _epilogue():
            p = acc_ref[...]
            h_agg = p[:, 0:d] + h0a_ref[...]
            h_i = theta * jnp.dot(h_agg, w_ref[...],
                                  preferred_element_type=jnp.float32)
            h_i = h_i + (1.0 - theta) * h_agg

            mu = p[:, d:2 * d]
            s = p[:, 2 * d:3 * d]
            s = jnp.where(s == 0.0, 1e-16, s)
            sig = jnp.sqrt(s)
            g3 = p[:, 3 * d:4 * d]
            g3 = jnp.where(g3 == 0.0, 1e-16, g3)
            a3 = jnp.abs(g3) ** (1.0 / 3.0)
            m3 = jnp.where(g3 < 0, -a3, a3)
            g4 = p[:, 4 * d:5 * d]
            g4 = jnp.where(g4 == 0.0, 1e-16, g4)
            a4 = jnp.abs(g4) ** 0.25
            m4 = jnp.where(g4 < 0, -a4, a4)

            wt = watt_ref[0:d, :]
            wb = watt_ref[d:2 * d, :]
            hw = jnp.dot(h_i, wb, preferred_element_type=jnp.float32)
            moms = (mu, sig, m3, m4)
            es = [
                jnp.dot(m, wt, preferred_element_type=jnp.float32) + hw
                for m in moms
            ]
            es = [jnp.where(e > 0, e, jnp.exp(e) - 1.0) for e in es]
            emax = jnp.maximum(jnp.maximum(es[0], es[1]),
                               jnp.maximum(es[2], es[3]))
            ws = [jnp.exp(e - emax) for e in es]
            denom = ws[0] + ws[1] + ws[2] + ws[3]
            h_m = (moms[0] * ws[0] + moms[1] * ws[1]
                   + moms[2] * ws[2] + moms[3] * ws[3]) / denom
            out_ref[...] = (1.0 - beta) * h_i + beta * h_m

    grid = (_cdiv(n, _BM), nk)
    out = pl.pallas_call(
        body,
        grid=grid,
        in_specs=[
            pl.BlockSpec((_BM, _BK), lambda i, k: (i, k)),
            pl.BlockSpec((kpad, nd * d), lambda i, k: (0, 0)),
            pl.BlockSpec((_BM, d), lambda i, k: (i, 0)),
            pl.BlockSpec((d, d), lambda i, k: (0, 0)),
            pl.BlockSpec((2 * d, d), lambda i, k: (0, 0)),
        ],
        out_specs=pl.BlockSpec((_BM, d), lambda i, k: (i, 0)),
        out_shape=jax.ShapeDtypeStruct((n, d), jnp.float32),
        scratch_shapes=[pltpu.VMEM((_BM, nd * d), jnp.float32)],
        compiler_params=pltpu.CompilerParams(
            dimension_semantics=("parallel", "arbitrary")),
    )(adj, x, h0a, weight, w_att)
    return out


# speedup vs baseline: 3.3687x; 1.3750x over previous
import math

import jax
import jax.numpy as jnp
from jax.experimental import pallas as pl
from jax.experimental.pallas import tpu as pltpu

_MOMENT = 4
_BM = 256
_LANE = 128


def _cdiv(a, b):
    return -(-a // b)


def kernel(input, adj, h0, weight, w_att, lamda, alpha, l):
    n, d = input.shape
    nd = _MOMENT + 1
    alpha = jnp.asarray(alpha, jnp.float32)

    h0_2 = h0 * h0
    x = jnp.concatenate(
        [(1.0 - alpha) * input, h0, h0_2, h0_2 * h0, h0_2 * h0_2], axis=1
    ).astype(jnp.bfloat16)
    h0a = alpha * h0

    theta = math.log(1.5)
    beta = 0.9
    kw = _cdiv(n, _LANE) * _LANE

    def body(adj_ref, x_ref, h0a_ref, w_ref, watt_ref, out_ref):
        a = adj_ref[:, 0:n].astype(jnp.bfloat16)
        p = jnp.dot(a, x_ref[...], preferred_element_type=jnp.float32)

        h_agg = p[:, 0:d] + h0a_ref[...]
        h_i = theta * jnp.dot(h_agg, w_ref[...],
                              preferred_element_type=jnp.float32)
        h_i = h_i + (1.0 - theta) * h_agg

        mu = p[:, d:2 * d]
        s = p[:, 2 * d:3 * d]
        s = jnp.where(s == 0.0, 1e-16, s)
        sig = jnp.sqrt(s)
        g3 = p[:, 3 * d:4 * d]
        g3 = jnp.where(g3 == 0.0, 1e-16, g3)
        a3 = jnp.abs(g3) ** (1.0 / 3.0)
        m3 = jnp.where(g3 < 0, -a3, a3)
        g4 = p[:, 4 * d:5 * d]
        g4 = jnp.where(g4 == 0.0, 1e-16, g4)
        a4 = jnp.abs(g4) ** 0.25
        m4 = jnp.where(g4 < 0, -a4, a4)

        wt = watt_ref[0:d, :]
        wb = watt_ref[d:2 * d, :]
        hw = jnp.dot(h_i, wb, preferred_element_type=jnp.float32)
        moms = (mu, sig, m3, m4)
        es = [
            jnp.dot(m, wt, preferred_element_type=jnp.float32) + hw
            for m in moms
        ]
        es = [jnp.where(e > 0, e, jnp.exp(e) - 1.0) for e in es]
        emax = jnp.maximum(jnp.maximum(es[0], es[1]),
                           jnp.maximum(es[2], es[3]))
        ws = [jnp.exp(e - emax) for e in es]
        denom = ws[0] + ws[1] + ws[2] + ws[3]
        h_m = (moms[0] * ws[0] + moms[1] * ws[1]
               + moms[2] * ws[2] + moms[3] * ws[3]) / denom
        out_ref[...] = (1.0 - beta) * h_i + beta * h_m

    grid = (_cdiv(n, _BM),)
    out = pl.pallas_call(
        body,
        grid=grid,
        in_specs=[
            pl.BlockSpec((_BM, kw), lambda i: (i, 0)),
            pl.BlockSpec((n, nd * d), lambda i: (0, 0)),
            pl.BlockSpec((_BM, d), lambda i: (i, 0)),
            pl.BlockSpec((d, d), lambda i: (0, 0)),
            pl.BlockSpec((2 * d, d), lambda i: (0, 0)),
        ],
        out_specs=pl.BlockSpec((_BM, d), lambda i: (i, 0)),
        out_shape=jax.ShapeDtypeStruct((n, d), jnp.float32),
        compiler_params=pltpu.CompilerParams(
            dimension_semantics=("parallel",)),
    )(adj, x, h0a, weight, w_att)
    return out


# speedup vs baseline: 3.3712x; 1.0007x over previous
import math

import jax
import jax.numpy as jnp
from jax.experimental import pallas as pl
from jax.experimental.pallas import tpu as pltpu

_MOMENT = 4
_BM = 384
_LANE = 128


def _cdiv(a, b):
    return -(-a // b)


def kernel(input, adj, h0, weight, w_att, lamda, alpha, l):
    n, d = input.shape
    nd = _MOMENT + 1
    alpha = jnp.asarray(alpha, jnp.float32)

    h0_2 = h0 * h0
    x = jnp.concatenate(
        [(1.0 - alpha) * input, h0, h0_2, h0_2 * h0, h0_2 * h0_2], axis=1
    ).astype(jnp.bfloat16)
    h0a = alpha * h0

    theta = math.log(1.5)
    beta = 0.9
    kw = _cdiv(n, _LANE) * _LANE

    def body(adj_ref, x_ref, h0a_ref, w_ref, watt_ref, out_ref):
        a = adj_ref[:, 0:n].astype(jnp.bfloat16)
        p = jnp.dot(a, x_ref[...], preferred_element_type=jnp.float32)

        h_agg = p[:, 0:d] + h0a_ref[...]
        h_i = theta * jnp.dot(h_agg, w_ref[...],
                              preferred_element_type=jnp.float32)
        h_i = h_i + (1.0 - theta) * h_agg

        mu = p[:, d:2 * d]
        s = p[:, 2 * d:3 * d]
        s = jnp.where(s == 0.0, 1e-16, s)
        sig = jnp.sqrt(s)
        g3 = p[:, 3 * d:4 * d]
        g3 = jnp.where(g3 == 0.0, 1e-16, g3)
        a3 = jnp.abs(g3) ** (1.0 / 3.0)
        m3 = jnp.where(g3 < 0, -a3, a3)
        g4 = p[:, 4 * d:5 * d]
        g4 = jnp.where(g4 == 0.0, 1e-16, g4)
        a4 = jnp.abs(g4) ** 0.25
        m4 = jnp.where(g4 < 0, -a4, a4)

        wt = watt_ref[0:d, :]
        wb = watt_ref[d:2 * d, :]
        hw = jnp.dot(h_i, wb, preferred_element_type=jnp.float32)
        moms = (mu, sig, m3, m4)
        es = [
            jnp.dot(m, wt, preferred_element_type=jnp.float32) + hw
            for m in moms
        ]
        es = [jnp.where(e > 0, e, jnp.exp(e) - 1.0) for e in es]
        emax = jnp.maximum(jnp.maximum(es[0], es[1]),
                           jnp.maximum(es[2], es[3]))
        ws = [jnp.exp(e - emax) for e in es]
        denom = ws[0] + ws[1] + ws[2] + ws[3]
        h_m = (moms[0] * ws[0] + moms[1] * ws[1]
               + moms[2] * ws[2] + moms[3] * ws[3]) / denom
        out_ref[...] = (1.0 - beta) * h_i + beta * h_m

    grid = (_cdiv(n, _BM),)
    out = pl.pallas_call(
        body,
        grid=grid,
        in_specs=[
            pl.BlockSpec((_BM, kw), lambda i: (i, 0)),
            pl.BlockSpec((n, nd * d), lambda i: (0, 0)),
            pl.BlockSpec((_BM, d), lambda i: (i, 0)),
            pl.BlockSpec((d, d), lambda i: (0, 0)),
            pl.BlockSpec((2 * d, d), lambda i: (0, 0)),
        ],
        out_specs=pl.BlockSpec((_BM, d), lambda i: (i, 0)),
        out_shape=jax.ShapeDtypeStruct((n, d), jnp.float32),
        compiler_params=pltpu.CompilerParams(
            dimension_semantics=("parallel",)),
    )(adj, x, h0a, weight, w_att)
    return out
